# Initial kernel scaffold; baseline (speedup 1.0000x reference)
import jax
import jax.numpy as jnp
from jax.experimental import pallas as pl


def kernel(x, edge_index, p_e, W_e, b_e):
    raise NotImplementedError("write your pallas kernel here")


# speedup vs baseline: 27.0246x; 27.0246x over previous
import functools

import jax
import jax.numpy as jnp
from jax import lax
from jax.experimental import pallas as pl
from jax.experimental.pallas import tpu as pltpu
from jax.experimental.pallas import tpu_sc as plsc


def _tc_node_scalars(x, w2d):
    n, d = x.shape

    def body(x_ref, w_ref, st_ref):
        st_ref[...] = lax.dot_general(
            w_ref[...], x_ref[...],
            dimension_numbers=(((1,), (1,)), ((), ())),
            preferred_element_type=jnp.float32,
        )

    return pl.pallas_call(
        body,
        out_shape=jax.ShapeDtypeStruct((2, n), jnp.float32),
    )(x, w2d)


def _sc_edge_sigmoid(st, src, dst, p_arr, b_arr):
    n = st.shape[1]
    e = src.shape[0]
    info = plsc.get_sparse_core_info()
    nc, ns, L = info.num_cores, info.num_subcores, info.num_lanes
    nw = nc * ns
    assert e % (nw * L) == 0
    epw = e // nw
    mesh = plsc.VectorSubcoreMesh(core_axis_name="c", subcore_axis_name="s")

    @functools.partial(
        pl.kernel,
        mesh=mesh,
        compiler_params=pltpu.CompilerParams(needs_layout_passes=False),
        out_type=(
            jax.ShapeDtypeStruct((e,), jnp.float32),
            jax.ShapeDtypeStruct((e,), jnp.float32),
        ),
        scratch_types=[
            pltpu.VMEM((n,), jnp.float32),
            pltpu.VMEM((n,), jnp.float32),
            pltpu.VMEM((epw,), jnp.int32),
            pltpu.VMEM((epw,), jnp.int32),
            pltpu.VMEM((epw,), jnp.float32),
            pltpu.VMEM((epw,), jnp.float32),
            pltpu.VMEM((L,), jnp.float32),
            pltpu.VMEM((L,), jnp.float32),
        ],
    )
    def k(st_hbm, src_hbm, dst_hbm, p_hbm, b_hbm, wp_hbm, beta_hbm,
          s_v, t_v, src_v, dst_v, wp_v, beta_v, p_v, b_v):
        wid = lax.axis_index("s") * nc + lax.axis_index("c")
        base = wid * epw
        pltpu.sync_copy(st_hbm.at[0], s_v)
        pltpu.sync_copy(st_hbm.at[1], t_v)
        pltpu.sync_copy(src_hbm.at[pl.ds(base, epw)], src_v)
        pltpu.sync_copy(dst_hbm.at[pl.ds(base, epw)], dst_v)
        pltpu.sync_copy(p_hbm, p_v)
        pltpu.sync_copy(b_hbm, b_v)
        p = p_v[...]
        b = b_v[...]

        def body(i, carry):
            off = i * L
            si = src_v[pl.ds(off, L)]
            di = dst_v[pl.ds(off, L)]
            a = plsc.load_gather(s_v, [si])
            c = plsc.load_gather(t_v, [di])
            z = a + c + b
            bet = 1.0 / (1.0 + jnp.exp(-z))
            beta_v[pl.ds(off, L)] = bet
            wp_v[pl.ds(off, L)] = 1.0 + bet * p
            return carry

        lax.fori_loop(0, epw // L, body, 0)
        pltpu.sync_copy(wp_v, wp_hbm.at[pl.ds(base, epw)])
        pltpu.sync_copy(beta_v, beta_hbm.at[pl.ds(base, epw)])

    return k(st, src, dst, p_arr, b_arr)


def kernel(x, edge_index, p_e, W_e, b_e):
    n, d = x.shape
    w2d = W_e.reshape(2, d)
    st = _tc_node_scalars(x, w2d)
    src = edge_index[0]
    dst = edge_index[1]
    L = plsc.get_sparse_core_info().num_lanes
    p_arr = jnp.full((L,), p_e, dtype=jnp.float32)
    b_arr = jnp.full((L,), b_e, dtype=jnp.float32)
    w_p, beta = _sc_edge_sigmoid(st, src, dst, p_arr, b_arr)
    return (w_p, beta)


# speedup vs baseline: 64.8749x; 2.4006x over previous
import functools

import jax
import jax.numpy as jnp
from jax import lax
from jax.experimental import pallas as pl
from jax.experimental.pallas import tpu as pltpu
from jax.experimental.pallas import tpu_sc as plsc


def _tc_node_scalars(x, w2d, p_e, b_e, L):
    n, d = x.shape

    def body(p_ref, b_ref, x_ref, w_ref, st_ref, scal_ref):
        st_ref[...] = lax.dot_general(
            w_ref[...], x_ref[...],
            dimension_numbers=(((1,), (1,)), ((), ())),
            preferred_element_type=jnp.float32,
        )
        scal_ref[...] = jnp.concatenate(
            [jnp.full((L,), p_ref[0], jnp.float32),
             jnp.full((L,), b_ref[0], jnp.float32)]
        )

    return pl.pallas_call(
        body,
        in_specs=[
            pl.BlockSpec(memory_space=pltpu.SMEM),
            pl.BlockSpec(memory_space=pltpu.SMEM),
            pl.BlockSpec((n, d), lambda: (0, 0)),
            pl.BlockSpec((2, d), lambda: (0, 0)),
        ],
        out_specs=[
            pl.BlockSpec((2, n), lambda: (0, 0)),
            pl.BlockSpec((2 * L,), lambda: (0,)),
        ],
        out_shape=[
            jax.ShapeDtypeStruct((2, n), jnp.float32),
            jax.ShapeDtypeStruct((2 * L,), jnp.float32),
        ],
    )(p_e.reshape(1), b_e.reshape(1), x, w2d)


def _sc_edge_sigmoid(st, edge_index, scal):
    n = st.shape[1]
    e = edge_index.shape[1]
    info = plsc.get_sparse_core_info()
    nc, ns, L = info.num_cores, info.num_subcores, info.num_lanes
    nw = nc * ns
    assert e % (nw * L) == 0
    epw = e // nw
    tile = 512
    cover = (epw + tile - 1) // tile * tile + tile
    assert e % tile == 0 and cover <= e
    mesh = plsc.VectorSubcoreMesh(core_axis_name="c", subcore_axis_name="s")

    splitA = (epw // 2 + (cover - epw) + tile - 1) // tile * tile
    splitB = cover - splitA

    @functools.partial(
        pl.kernel,
        mesh=mesh,
        compiler_params=pltpu.CompilerParams(needs_layout_passes=False),
        out_type=(
            jax.ShapeDtypeStruct((e,), jnp.float32),
            jax.ShapeDtypeStruct((e,), jnp.float32),
        ),
        scratch_types=[
            pltpu.VMEM((n,), jnp.float32),
            pltpu.VMEM((n,), jnp.float32),
            pltpu.VMEM((2, cover), jnp.int32),
            pltpu.VMEM((epw,), jnp.float32),
            pltpu.VMEM((epw,), jnp.float32),
            pltpu.VMEM((L,), jnp.float32),
            pltpu.VMEM((L,), jnp.float32),
            pltpu.VMEM_SHARED((n,), jnp.float32),
            pltpu.VMEM_SHARED((n,), jnp.float32),
            pltpu.SemaphoreType.DMA,
            pltpu.SemaphoreType.DMA,
            pltpu.SemaphoreType.DMA,
            pltpu.SemaphoreType.DMA,
            pltpu.SemaphoreType.DMA,
            pltpu.SemaphoreType.DMA,
            pltpu.SemaphoreType.DMA,
            pltpu.SemaphoreType.DMA,
        ],
    )
    def k(st_hbm, ei_hbm, scal_hbm, wp_hbm, beta_hbm,
          s_v, t_v, ei_v, wp_v, beta_v, p_v, b_v, s_sh, t_sh,
          sem0, sem1, sem2, sem3, sem4, sem5, sem6, sem7):
        sid = lax.axis_index("s")
        wid = sid * nc + lax.axis_index("c")
        base = wid * epw
        lo = jnp.minimum(base // tile * tile, e - cover)
        lo = pl.multiple_of(lo, tile)
        local = base - lo
        c2a = pltpu.async_copy(ei_hbm.at[:, pl.ds(lo, splitA)],
                               ei_v.at[:, pl.ds(0, splitA)], sem2)
        c2b = pltpu.async_copy(ei_hbm.at[:, pl.ds(lo + splitA, splitB)],
                               ei_v.at[:, pl.ds(splitA, splitB)], sem7)

        @pl.when(sid == 0)
        def _():
            i0 = pltpu.async_copy(st_hbm.at[0], s_sh, sem0)
            i1 = pltpu.async_copy(st_hbm.at[1], t_sh, sem1)
            i0.wait()
            i1.wait()

        plsc.subcore_barrier()
        c0 = pltpu.async_copy(s_sh, s_v, sem0)
        c1 = pltpu.async_copy(t_sh, t_v, sem1)
        pltpu.sync_copy(scal_hbm.at[pl.ds(0, L)], p_v)
        pltpu.sync_copy(scal_hbm.at[pl.ds(L, L)], b_v)
        p = p_v[...]
        b = b_v[...]
        c0.wait()
        c1.wait()
        c2a.wait()

        def run_chunk(lo_e, hi_e):
            @plsc.parallel_loop(lo_e, hi_e, L, unroll=8)
            def body(off):
                si = ei_v[0, pl.ds(local + off, L)]
                di = ei_v[1, pl.ds(local + off, L)]
                a = plsc.load_gather(s_v, [si])
                c = plsc.load_gather(t_v, [di])
                z = a + c + b
                bet = 1.0 / (1.0 + jnp.exp(-z))
                beta_v[pl.ds(off, L)] = bet
                wp_v[pl.ds(off, L)] = 1.0 + bet * p

        h0 = (epw // 2) // L * L
        h1 = epw - h0
        run_chunk(0, h0)
        o0 = pltpu.async_copy(wp_v.at[pl.ds(0, h0)],
                              wp_hbm.at[pl.ds(base, h0)], sem3)
        o1 = pltpu.async_copy(beta_v.at[pl.ds(0, h0)],
                              beta_hbm.at[pl.ds(base, h0)], sem4)
        c2b.wait()
        run_chunk(h0, epw)
        o2 = pltpu.async_copy(wp_v.at[pl.ds(h0, h1)],
                              wp_hbm.at[pl.ds(base + h0, h1)], sem5)
        o3 = pltpu.async_copy(beta_v.at[pl.ds(h0, h1)],
                              beta_hbm.at[pl.ds(base + h0, h1)], sem6)
        o0.wait()
        o1.wait()
        o2.wait()
        o3.wait()

    return k(st, edge_index, scal)


def kernel(x, edge_index, p_e, W_e, b_e):
    n, d = x.shape
    w2d = W_e.reshape(2, d)
    L = plsc.get_sparse_core_info().num_lanes
    st, scal = _tc_node_scalars(x, w2d,
                                jnp.asarray(p_e, jnp.float32),
                                jnp.asarray(b_e, jnp.float32), L)
    w_p, beta = _sc_edge_sigmoid(st, edge_index, scal)
    return (w_p, beta)
